# Initial kernel scaffold; baseline (speedup 1.0000x reference)
import jax
import jax.numpy as jnp
from jax.experimental import pallas as pl


def kernel(positions, neighbor_vectors, first_atom):
    raise NotImplementedError("write your pallas kernel here")


# speedup vs baseline: 1.0471x; 1.0471x over previous
import functools

import jax
import jax.numpy as jnp
from jax import lax
from jax.experimental import pallas as pl
from jax.experimental.pallas import tpu as pltpu
from jax.experimental.pallas import tpu_sc as plsc

_CUTOFF = 5.0
_K = 8
_E = 3_200_000
_N = 100_000
_NC = 2
_NS = 16
_NW = _NC * _NS
_C = 1024
_CHUNKS = _E // _C
_RPT = (_N // _NS) // 8 * 8
_REM = _N - _NS * _RPT
_GROUPS = _C // 16


def _seg_body(nv_hbm, fa_hbm, z_hbm, out_hbm, nvb, idxb, rows, zbuf, facc):
    c = lax.axis_index("c")
    s = lax.axis_index("s")
    w = s * _NC + c

    pltpu.sync_copy(z_hbm, zbuf)
    pltpu.sync_copy(zbuf.at[pl.ds(0, _RPT)], facc.at[pl.ds(s * _RPT, _RPT)])

    @pl.when(s == _NS - 1)
    def _():
        pltpu.sync_copy(
            zbuf.at[pl.ds(0, _REM)], facc.at[pl.ds(_NS * _RPT, _REM)]
        )

    plsc.subcore_barrier()

    ii = lax.iota(jnp.int32, 16)
    ones = jnp.full((16,), 1.0, jnp.float32)

    def chunk_body(t, _):
        m = w + t * _NW

        pltpu.sync_copy(nv_hbm.at[pl.ds(m * (3 * _C), 3 * _C)], nvb)
        pltpu.sync_copy(fa_hbm.at[pl.ds(m * (_C // 128), _C // 128)], idxb)

        def group_body(g, _):
            base = g * 48
            ix = base + 3 * ii
            x = plsc.load_gather(nvb, [ix])
            y = plsc.load_gather(nvb, [ix + 1])
            z = plsc.load_gather(nvb, [ix + 2])
            ss = x * x + y * y + z * z
            ssc = jnp.maximum(ss, jnp.float32(1e-37))
            t32 = plsc.bitcast(ssc, jnp.int32)
            t32 = jnp.int32(0x5F3759DF) - (t32 >> 1)
            q = plsc.bitcast(t32, jnp.float32)
            h = ssc * jnp.float32(0.5)
            q = q * (jnp.float32(1.5) - h * q * q)
            q = q * (jnp.float32(1.5) - h * q * q)
            q = q * (jnp.float32(1.5) - h * q * q)
            r = ss * q * jnp.float32(1.0 / _CUTOFF)

            erow = g * 16 + ii
            plsc.store_scatter(rows, [erow, jnp.full((16,), 0, jnp.int32)], ones)
            pk = r
            for k in range(1, _K):
                plsc.store_scatter(
                    rows, [erow, jnp.full((16,), k, jnp.int32)], pk
                )
                pk = pk * r
            return ()

        lax.fori_loop(0, _GROUPS, group_body, ())

        for j in range(_C // 128):
            pltpu.sync_copy(
                rows.at[pl.ds(j * 128, 128)], facc.at[idxb.at[j]], add=True
            )
        return ()

    n_chunks = jnp.where(w < _CHUNKS % _NW, _CHUNKS // _NW + 1, _CHUNKS // _NW)
    lax.fori_loop(0, n_chunks, chunk_body, ())

    plsc.subcore_barrier()
    r0 = c * _N + s * _RPT
    pltpu.sync_copy(facc.at[pl.ds(s * _RPT, _RPT)], zbuf.at[pl.ds(0, _RPT)])
    pltpu.sync_copy(zbuf.at[pl.ds(0, _RPT)], out_hbm.at[pl.ds(r0, _RPT)])

    @pl.when(s == _NS - 1)
    def _():
        pltpu.sync_copy(
            facc.at[pl.ds(_NS * _RPT, _REM)], zbuf.at[pl.ds(_RPT, _REM)]
        )
        pltpu.sync_copy(
            zbuf.at[pl.ds(_RPT, _REM)],
            out_hbm.at[pl.ds(c * _N + _NS * _RPT, _REM)],
        )


_seg_kernel = functools.partial(
    pl.kernel,
    out_type=jax.ShapeDtypeStruct((_NC * _N, _K), jnp.float32),
    mesh=plsc.VectorSubcoreMesh(core_axis_name="c", subcore_axis_name="s"),
    compiler_params=pltpu.CompilerParams(
        needs_layout_passes=False, use_tc_tiling_on_sc=False
    ),
    scratch_types=[
        pltpu.VMEM((3 * _C,), jnp.float32),
        pltpu.VMEM((_C // 128, 128), jnp.int32),
        pltpu.VMEM((_C, _K), jnp.float32),
        pltpu.VMEM((_RPT + _REM, _K), jnp.float32),
        pltpu.VMEM_SHARED((_N, _K), jnp.float32),
    ],
)(_seg_body)


def kernel(positions, neighbor_vectors, first_atom):
    n = positions.shape[0]
    nv_flat = neighbor_vectors.reshape(-1)
    fa2d = first_atom.reshape(-1, 128)
    zeros_rows = jnp.zeros((_RPT + _REM, _K), jnp.float32)
    parts = _seg_kernel(nv_flat, fa2d, zeros_rows)
    features = parts[:n] + parts[n:]
    centered = features - jnp.mean(features)
    _, _, vh = jnp.linalg.svd(centered, full_matrices=False)
    return features @ vh[:3].T
